# baseline (speedup 1.0000x reference)
import functools

import jax
import jax.numpy as jnp
import numpy as np
from jax.experimental import pallas as pl
from jax.experimental.pallas import tpu as pltpu

H = 6
C = 32
D = H * C
G = 32


def _ln(x, g, b):
    m = jnp.mean(x, axis=-1, keepdims=True)
    v = jnp.mean((x - m) ** 2, axis=-1, keepdims=True)
    return (x - m) / jnp.sqrt(v + 1e-5) * g + b


def _tconv(x, edge_index, edge_attr, p):
    src = edge_index[0]
    dst = edge_index[1]
    n = x.shape[0]
    q = (x @ p['Wq'] + p['bq']).reshape(n, H, C)
    k = (x @ p['Wk'] + p['bk']).reshape(n, H, C)
    v = (x @ p['Wv'] + p['bv']).reshape(n, H, C)
    e = (edge_attr @ p['We']).reshape(-1, H, C)
    kj = k[src] + e
    vj = v[src] + e
    alpha = jnp.sum(q[dst] * kj, axis=-1) / np.sqrt(C)
    amax = jax.ops.segment_max(alpha, dst, num_segments=n)
    amax = jnp.where(jnp.isfinite(amax), amax, 0.0)
    ex = jnp.exp(alpha - amax[dst])
    den = jax.ops.segment_sum(ex, dst, num_segments=n)
    alpha = ex / (den[dst] + 1e-16)
    out = jax.ops.segment_sum(alpha[:, :, None] * vj, dst, num_segments=n)
    out = out.reshape(n, H * C)
    return out + x @ p['Ws'] + p['bs']


def _pool_kernel(h_ref, b_ref, sums_ref, cnt_ref):
    i = pl.program_id(0)

    @pl.when(i == 0)
    def _():
        sums_ref[...] = jnp.zeros_like(sums_ref)
        cnt_ref[...] = jnp.zeros_like(cnt_ref)

    hb = h_ref[...]
    bb = b_ref[0]
    onehot = (bb == jax.lax.broadcasted_iota(jnp.int32, (G, 1), 0)).astype(jnp.float32)
    sums_ref[...] += jnp.dot(onehot, hb, preferred_element_type=jnp.float32)
    cnt_ref[...] += jnp.broadcast_to(
        jnp.sum(onehot, axis=1, keepdims=True), cnt_ref.shape)


def _head_kernel(sums_ref, cnt_ref, ie_ref,
                 fciW_ref, fcib_ref, fcig_ref, fcibn_ref,
                 fc1W_ref, fc1b_ref, fc1g_ref, fc1bn_ref,
                 fc2W_ref, fc2b_ref, fc2g_ref, fc2bn_ref,
                 fc3W_ref, fc3b_ref, out_ref):
    cnt = cnt_ref[:, 0:1]
    gf = sums_ref[...] / jnp.maximum(cnt, 1.0)
    ie = ie_ref[...]
    fi = jax.nn.relu(_ln(jnp.dot(ie, fciW_ref[...],
                                 preferred_element_type=jnp.float32)
                         + fcib_ref[...], fcig_ref[...], fcibn_ref[...]))
    z = jnp.concatenate([gf, fi], axis=1)
    z = jax.nn.relu(_ln(jnp.dot(z, fc1W_ref[...],
                                preferred_element_type=jnp.float32)
                        + fc1b_ref[...], fc1g_ref[...], fc1bn_ref[...]))
    z = jax.nn.relu(_ln(jnp.dot(z, fc2W_ref[...],
                                preferred_element_type=jnp.float32)
                        + fc2b_ref[...], fc2g_ref[...], fc2bn_ref[...]))
    out_ref[...] = jnp.dot(z, fc3W_ref[...],
                           preferred_element_type=jnp.float32) + fc3b_ref[...]


def kernel(x, edge_index, edge_weight, batch, initial_energies, params):
    p = params
    h = x @ p['proj_W'] + p['proj_b']
    h1 = jax.nn.relu(_ln(_tconv(h, edge_index, edge_weight, p['t1']), p['ln1_g'], p['ln1_b']))
    h = h + h1
    h2 = jax.nn.relu(_ln(_tconv(h, edge_index, edge_weight, p['t2']), p['ln2_g'], p['ln2_b']))
    h = h + h2
    h3 = jax.nn.relu(_ln(_tconv(h, edge_index, edge_weight, p['t3']), p['ln3_g'], p['ln3_b']))
    h = h + h3

    n = h.shape[0]
    Bn = 2000
    grid = n // Bn
    sums, cnt = pl.pallas_call(
        _pool_kernel,
        grid=(grid,),
        in_specs=[
            pl.BlockSpec((Bn, D), lambda i: (i, 0)),
            pl.BlockSpec((1, 1, Bn), lambda i: (i, 0, 0)),
        ],
        out_specs=[
            pl.BlockSpec((G, D), lambda i: (0, 0)),
            pl.BlockSpec((G, 128), lambda i: (0, 0)),
        ],
        out_shape=[
            jax.ShapeDtypeStruct((G, D), jnp.float32),
            jax.ShapeDtypeStruct((G, 128), jnp.float32),
        ],
    )(h, batch.reshape(grid, 1, Bn))

    full = lambda a: pl.BlockSpec(a.shape, lambda: tuple(0 for _ in a.shape))
    ie = initial_energies.reshape(G, 1)
    head_args = [sums, cnt, ie,
                 p['fci_W'], p['fci_b'].reshape(1, D),
                 p['fci_g'].reshape(1, D), p['fci_bn'].reshape(1, D),
                 p['fc1_W'], p['fc1_b'].reshape(1, D),
                 p['fc1_g'].reshape(1, D), p['fc1_bn'].reshape(1, D),
                 p['fc2_W'], p['fc2_b'].reshape(1, D // 2),
                 p['fc2_g'].reshape(1, D // 2), p['fc2_bn'].reshape(1, D // 2),
                 p['fc3_W'], p['fc3_b'].reshape(1, 1)]
    out = pl.pallas_call(
        _head_kernel,
        in_specs=[full(a) for a in head_args],
        out_specs=full(jnp.zeros((G, 1))),
        out_shape=jax.ShapeDtypeStruct((G, 1), jnp.float32),
    )(*head_args)
    return out


# speedup vs baseline: 13.5886x; 13.5886x over previous
import functools

import jax
import jax.numpy as jnp
import numpy as np
from jax import lax
from jax.experimental import pallas as pl
from jax.experimental.pallas import tpu as pltpu
from jax.experimental.pallas import tpu_sc as plsc

N = 10000
E = 320000
H = 6
C = 32
D = H * C
G = 32
SCALE = 1.0 / np.sqrt(C)

NW = 32
SD = 40
CHR = 8
CH = SD * CHR
RW = 256
NCH = RW // CHR
EPW = RW * SD
EPAD = NW * EPW
NP = 10240
RT = NP // 16
GW = 256
SW = 128

BN = 1000
BE = 4096


def _proj_kernel(x_ref, w_ref, b_ref, o_ref):
    o_ref[...] = jnp.dot(x_ref[...], w_ref[...],
                         preferred_element_type=jnp.float32) + b_ref[...]


def _qkvs_kernel(h_ref, wqa_ref, bqa_ref, wk_ref, bk_ref, wv_ref, bv_ref,
                 ws_ref, bs_ref, qa_ref, k_ref, v_ref, dn_ref):
    hb = h_ref[...]
    qa_ref[...] = jnp.dot(hb, wqa_ref[...],
                          preferred_element_type=jnp.float32) + bqa_ref[...]
    k_ref[...] = jnp.dot(hb, wk_ref[...],
                         preferred_element_type=jnp.float32) + bk_ref[...]
    v_ref[...] = jnp.dot(hb, wv_ref[...],
                         preferred_element_type=jnp.float32) + bv_ref[...]
    dn_ref[...] = jnp.dot(hb, ws_ref[...],
                          preferred_element_type=jnp.float32) + bs_ref[...]


def _alpha_kernel(qd_ref, ks_ref, w_ref, s16_ref, t16_ref, a_ref, m_ref):
    i = pl.program_id(0)
    qd = qd_ref[...]
    prod = qd * ks_ref[...]
    a1 = jnp.dot(prod, s16_ref[...], preferred_element_type=jnp.float32)
    qwe = jnp.dot(qd, t16_ref[...], preferred_element_type=jnp.float32)
    alpha = a1 + w_ref[...] * qwe
    a_ref[...] = alpha

    @pl.when(i == 0)
    def _():
        m_ref[...] = jnp.full_like(m_ref, -1e30)

    m_ref[...] = jnp.maximum(m_ref[...], jnp.max(alpha, axis=0, keepdims=True))


def _exwv_kernel(a_ref, m_ref, pmask_ref, vs_ref, w_ref, msk_ref,
                 b6p_ref, p2_ref, ra_ref, rb_ref, rc_ref, sa_ref, sb_ref):
    ex = jnp.exp(a_ref[...] - (m_ref[...] + pmask_ref[...])) * msk_ref[...]
    ds = ex + jnp.dot(ex * w_ref[...], p2_ref[...],
                      preferred_element_type=jnp.float32)
    exb = jnp.dot(ex, b6p_ref[...], preferred_element_type=jnp.float32)
    wv = vs_ref[...] * exb
    sa_ref[...] = jnp.dot(wv, ra_ref[...], preferred_element_type=jnp.float32)
    sb_ref[...] = (jnp.dot(wv, rb_ref[...], preferred_element_type=jnp.float32)
                   + jnp.dot(ds, rc_ref[...],
                             preferred_element_type=jnp.float32))


def _combine_kernel(dn_ref, a0_ref, a1_ref, b0_ref, b1_ref, hin_ref,
                    ea_ref, eb_ref, ec_ref, b6_ref, p2_ref, w2e_ref,
                    dmask_ref, g_ref, b_ref, o_ref):
    asum = a0_ref[...] + a1_ref[...]
    bsum = b0_ref[...] + b1_ref[...]
    num = (jnp.dot(asum, ea_ref[...], preferred_element_type=jnp.float32)
           + jnp.dot(bsum, eb_ref[...], preferred_element_type=jnp.float32))
    r = jnp.dot(bsum, ec_ref[...], preferred_element_type=jnp.float32)
    den192 = jnp.dot(r, b6_ref[...], preferred_element_type=jnp.float32)
    t1 = num / (den192 + 1e-16)
    den_shift = jnp.dot(r, p2_ref[...], preferred_element_type=jnp.float32)
    rdiv = r / (den_shift + dmask_ref[...] + 1e-16)
    t2 = jnp.dot(rdiv, w2e_ref[...], preferred_element_type=jnp.float32)
    y = dn_ref[...] + t1 + t2
    m = jnp.mean(y, axis=-1, keepdims=True)
    v = jnp.mean((y - m) ** 2, axis=-1, keepdims=True)
    yn = (y - m) / jnp.sqrt(v + 1e-5) * g_ref[...] + b_ref[...]
    o_ref[...] = hin_ref[...] + jax.nn.relu(yn)


def _pool_kernel(h_ref, b_ref, sums_ref, cnt_ref):
    i = pl.program_id(0)

    @pl.when(i == 0)
    def _():
        sums_ref[...] = jnp.zeros_like(sums_ref)
        cnt_ref[...] = jnp.zeros_like(cnt_ref)

    hb = h_ref[...]
    bb = b_ref[0]
    onehot = (bb == lax.broadcasted_iota(jnp.int32, (G, 1), 0)).astype(jnp.float32)
    sums_ref[...] += jnp.dot(onehot, hb, preferred_element_type=jnp.float32)
    cnt_ref[...] += jnp.broadcast_to(
        jnp.sum(onehot, axis=1, keepdims=True), cnt_ref.shape)


def _lnp(x, g, b):
    m = jnp.mean(x, axis=-1, keepdims=True)
    v = jnp.mean((x - m) ** 2, axis=-1, keepdims=True)
    return (x - m) / jnp.sqrt(v + 1e-5) * g + b


def _head_kernel(sums_ref, cnt_ref, ie_ref,
                 fciW_ref, fcib_ref, fcig_ref, fcibn_ref,
                 fc1W_ref, fc1b_ref, fc1g_ref, fc1bn_ref,
                 fc2W_ref, fc2b_ref, fc2g_ref, fc2bn_ref,
                 fc3W_ref, fc3b_ref, out_ref):
    cnt = cnt_ref[:, 0:1]
    gf = sums_ref[...] / jnp.maximum(cnt, 1.0)
    fi = jax.nn.relu(_lnp(jnp.dot(ie_ref[...], fciW_ref[...],
                                  preferred_element_type=jnp.float32)
                          + fcib_ref[...], fcig_ref[...], fcibn_ref[...]))
    z = jnp.concatenate([gf, fi], axis=1)
    z = jax.nn.relu(_lnp(jnp.dot(z, fc1W_ref[...],
                                 preferred_element_type=jnp.float32)
                         + fc1b_ref[...], fc1g_ref[...], fc1bn_ref[...]))
    z = jax.nn.relu(_lnp(jnp.dot(z, fc2W_ref[...],
                                 preferred_element_type=jnp.float32)
                         + fc2b_ref[...], fc2g_ref[...], fc2bn_ref[...]))
    out_ref[...] = jnp.dot(z, fc3W_ref[...],
                           preferred_element_type=jnp.float32) + fc3b_ref[...]


def _sc_gather_body(qa_hbm, k_hbm, v_hbm, src_hbm, dst_hbm,
                    qd_out, ks_out, vs_out, didx, sidx, buf, sem):
    c = lax.axis_index("c")
    s = lax.axis_index("s")
    w = c * 16 + s
    row0 = w * RW

    @pl.loop(0, NCH)
    def _(i):
        irow = row0 + i * CHR
        base = w * EPW + i * CH
        pltpu.sync_copy(dst_hbm.at[pl.ds(irow, CHR)], didx)
        pltpu.sync_copy(src_hbm.at[pl.ds(irow, CHR)], sidx)
        for table, idx, out in ((qa_hbm, didx, qd_out),
                                (k_hbm, sidx, ks_out),
                                (v_hbm, sidx, vs_out)):
            cps = [pltpu.async_copy(table.at[idx.at[j]],
                                    buf.at[pl.ds(j * SD, SD)], sem)
                   for j in range(CHR)]
            for cp in cps:
                cp.wait()
            pltpu.sync_copy(buf, out.at[pl.ds(base, CH)])


def _sc_scatter_body(rows_hbm, dst_hbm, zeros_hbm, out_hbm,
                     didx, rbuf, table, sem):
    c = lax.axis_index("c")
    s = lax.axis_index("s")
    w = c * 16 + s
    row0 = w * RW
    pltpu.sync_copy(zeros_hbm.at[pl.ds(s * RT, RT)],
                    table.at[pl.ds(s * RT, RT)])
    plsc.subcore_barrier()

    @pl.loop(0, NCH)
    def _(i):
        irow = row0 + i * CHR
        base = w * EPW + i * CH
        pltpu.sync_copy(dst_hbm.at[pl.ds(irow, CHR)], didx)
        pltpu.sync_copy(rows_hbm.at[pl.ds(base, CH)], rbuf)
        cps = [pltpu.async_copy(rbuf.at[pl.ds(j * SD, SD)],
                                table.at[didx.at[j]], sem, add=True)
               for j in range(CHR)]
        for cp in cps:
            cp.wait()

    plsc.subcore_barrier()
    pltpu.sync_copy(table.at[pl.ds(s * RT, RT)],
                    out_hbm.at[pl.ds(c * NP + s * RT, RT)])


@functools.cache
def _sc_kernels():
    mesh = plsc.VectorSubcoreMesh(core_axis_name="c", subcore_axis_name="s")
    gather = pl.kernel(
        _sc_gather_body,
        out_type=[jax.ShapeDtypeStruct((EPAD, GW), jnp.float32),
                  jax.ShapeDtypeStruct((EPAD, GW), jnp.float32),
                  jax.ShapeDtypeStruct((EPAD, GW), jnp.float32)],
        mesh=mesh,
        scratch_types=[pltpu.VMEM((CHR, SD), jnp.int32),
                       pltpu.VMEM((CHR, SD), jnp.int32),
                       pltpu.VMEM((CH, GW), jnp.float32),
                       pltpu.SemaphoreType.DMA],
    )
    scatter = pl.kernel(
        _sc_scatter_body,
        out_type=jax.ShapeDtypeStruct((2 * NP, SW), jnp.float32),
        mesh=mesh,
        scratch_types=[pltpu.VMEM((CHR, SD), jnp.int32),
                       pltpu.VMEM((CH, SW), jnp.float32),
                       pltpu.VMEM_SHARED((NP, SW), jnp.float32),
                       pltpu.SemaphoreType.DMA],
    )
    return gather, scatter


def _static_mats():
    s16 = np.zeros((GW, 16), np.float32)
    for j in range(D):
        s16[j, j // C] = 1.0
    t16 = np.zeros((GW, 16), np.float32)
    for h in range(H):
        t16[D + h, h] = 1.0
    b6 = np.zeros((16, D), np.float32)
    for h in range(H):
        b6[h, h * C:(h + 1) * C] = 1.0
    b6p = np.zeros((16, GW), np.float32)
    b6p[:, :D] = b6
    p2 = np.zeros((16, 16), np.float32)
    for h in range(H):
        p2[h, h + 6] = 1.0
    ra = np.zeros((GW, SW), np.float32)
    ra[:SW, :] = np.eye(SW, dtype=np.float32)
    rb = np.zeros((GW, SW), np.float32)
    rb[SW:D, :D - SW] = np.eye(D - SW, dtype=np.float32)
    rc = np.zeros((16, SW), np.float32)
    rc[:, D - SW:D - SW + 16] = np.eye(16, dtype=np.float32)
    ea = np.zeros((SW, D), np.float32)
    ea[:, :SW] = np.eye(SW, dtype=np.float32)
    eb = np.zeros((SW, D), np.float32)
    eb[:D - SW, SW:] = np.eye(D - SW, dtype=np.float32)
    ec = np.zeros((SW, 16), np.float32)
    ec[D - SW:D - SW + 16, :] = np.eye(16, dtype=np.float32)
    pmask = np.zeros((1, 16), np.float32)
    pmask[0, H:] = 1e30
    dmask = np.ones((1, 16), np.float32)
    dmask[0, H:2 * H] = 0.0
    return {k: jnp.asarray(v) for k, v in dict(
        s16=s16, t16=t16, b6=b6, b6p=b6p, p2=p2, ra=ra, rb=rb, rc=rc,
        ea=ea, eb=eb, ec=ec, pmask=pmask, dmask=dmask).items()}


_HEADMASK = (np.arange(H)[:, None] == np.arange(D)[None, :] // C
             ).astype(np.float32)


def _prep_tconv(p, ln_g, ln_b):
    we = p['We'].reshape(D)
    mask = (np.arange(D)[:, None] // C == np.arange(H)[None, :]).astype(np.float32)
    w2 = we[:, None] * mask
    padq = jnp.zeros((D, GW - D - H), jnp.float32)
    wqa = jnp.concatenate([p['Wq'] * SCALE, (p['Wq'] @ w2) * SCALE, padq], axis=1)
    bqa = jnp.concatenate([p['bq'] * SCALE, (p['bq'] @ w2) * SCALE,
                           jnp.zeros((GW - D - H,), jnp.float32)]).reshape(1, GW)
    padk = jnp.zeros((D, GW - D), jnp.float32)
    bpad = jnp.zeros((GW - D,), jnp.float32)
    w2e_rows = _HEADMASK * p['We'].reshape(1, D)
    w2e = jnp.concatenate([jnp.zeros((6, D), jnp.float32), w2e_rows,
                           jnp.zeros((4, D), jnp.float32)], axis=0)
    return {
        'Wqa': wqa, 'bqa': bqa,
        'Wk': jnp.concatenate([p['Wk'], padk], axis=1),
        'bk': jnp.concatenate([p['bk'], bpad]).reshape(1, GW),
        'Wv': jnp.concatenate([p['Wv'], padk], axis=1),
        'bv': jnp.concatenate([p['bv'], bpad]).reshape(1, GW),
        'Ws': p['Ws'], 'bs2': p['bs'].reshape(1, D),
        'ln_g': ln_g.reshape(1, D), 'ln_b': ln_b.reshape(1, D),
        'w2e': w2e,
    }


def _full_spec(a):
    return pl.BlockSpec(a.shape, lambda *_: tuple(0 for _ in a.shape))


def _run_layer(h, src2, dst2, ew, emask, tp, cm, z128):
    qa, k256, v256, dense = pl.pallas_call(
        _qkvs_kernel,
        grid=(N // BN,),
        in_specs=[pl.BlockSpec((BN, D), lambda i: (i, 0))] + [
            pl.BlockSpec(w.shape, lambda i: (0, 0)) for w in
            (tp['Wqa'], tp['bqa'], tp['Wk'], tp['bk'],
             tp['Wv'], tp['bv'], tp['Ws'], tp['bs2'])],
        out_specs=[pl.BlockSpec((BN, GW), lambda i: (i, 0)),
                   pl.BlockSpec((BN, GW), lambda i: (i, 0)),
                   pl.BlockSpec((BN, GW), lambda i: (i, 0)),
                   pl.BlockSpec((BN, D), lambda i: (i, 0))],
        out_shape=[jax.ShapeDtypeStruct((N, GW), jnp.float32),
                   jax.ShapeDtypeStruct((N, GW), jnp.float32),
                   jax.ShapeDtypeStruct((N, GW), jnp.float32),
                   jax.ShapeDtypeStruct((N, D), jnp.float32)],
    )(h, tp['Wqa'], tp['bqa'], tp['Wk'], tp['bk'],
      tp['Wv'], tp['bv'], tp['Ws'], tp['bs2'])

    sc_gather, sc_scatter = _sc_kernels()
    qd, ks, vs = sc_gather(qa, k256, v256, src2, dst2)

    alpha, m = pl.pallas_call(
        _alpha_kernel,
        grid=(EPAD // BE,),
        in_specs=[pl.BlockSpec((BE, GW), lambda i: (i, 0)),
                  pl.BlockSpec((BE, GW), lambda i: (i, 0)),
                  pl.BlockSpec((BE, 1), lambda i: (i, 0)),
                  pl.BlockSpec((GW, 16), lambda i: (0, 0)),
                  pl.BlockSpec((GW, 16), lambda i: (0, 0))],
        out_specs=[pl.BlockSpec((BE, 16), lambda i: (i, 0)),
                   pl.BlockSpec((1, 16), lambda i: (0, 0))],
        out_shape=[jax.ShapeDtypeStruct((EPAD, 16), jnp.float32),
                   jax.ShapeDtypeStruct((1, 16), jnp.float32)],
    )(qd, ks, ew, cm['s16'], cm['t16'])

    sa, sb = pl.pallas_call(
        _exwv_kernel,
        grid=(EPAD // BE,),
        in_specs=[pl.BlockSpec((BE, 16), lambda i: (i, 0)),
                  pl.BlockSpec((1, 16), lambda i: (0, 0)),
                  pl.BlockSpec((1, 16), lambda i: (0, 0)),
                  pl.BlockSpec((BE, GW), lambda i: (i, 0)),
                  pl.BlockSpec((BE, 1), lambda i: (i, 0)),
                  pl.BlockSpec((BE, 1), lambda i: (i, 0)),
                  pl.BlockSpec((16, GW), lambda i: (0, 0)),
                  pl.BlockSpec((16, 16), lambda i: (0, 0)),
                  pl.BlockSpec((GW, SW), lambda i: (0, 0)),
                  pl.BlockSpec((GW, SW), lambda i: (0, 0)),
                  pl.BlockSpec((16, SW), lambda i: (0, 0))],
        out_specs=[pl.BlockSpec((BE, SW), lambda i: (i, 0)),
                   pl.BlockSpec((BE, SW), lambda i: (i, 0))],
        out_shape=[jax.ShapeDtypeStruct((EPAD, SW), jnp.float32),
                   jax.ShapeDtypeStruct((EPAD, SW), jnp.float32)],
    )(alpha, m, cm['pmask'], vs, ew, emask,
      cm['b6p'], cm['p2'], cm['ra'], cm['rb'], cm['rc'])

    apart = sc_scatter(sa, dst2, z128)
    bpart = sc_scatter(sb, dst2, z128)

    a0 = lax.slice(apart, (0, 0), (N, SW))
    a1 = lax.slice(apart, (NP, 0), (NP + N, SW))
    b0 = lax.slice(bpart, (0, 0), (N, SW))
    b1 = lax.slice(bpart, (NP, 0), (NP + N, SW))

    h_out = pl.pallas_call(
        _combine_kernel,
        grid=(N // BN,),
        in_specs=[pl.BlockSpec((BN, D), lambda i: (i, 0)),
                  pl.BlockSpec((BN, SW), lambda i: (i, 0)),
                  pl.BlockSpec((BN, SW), lambda i: (i, 0)),
                  pl.BlockSpec((BN, SW), lambda i: (i, 0)),
                  pl.BlockSpec((BN, SW), lambda i: (i, 0)),
                  pl.BlockSpec((BN, D), lambda i: (i, 0)),
                  pl.BlockSpec((SW, D), lambda i: (0, 0)),
                  pl.BlockSpec((SW, D), lambda i: (0, 0)),
                  pl.BlockSpec((SW, 16), lambda i: (0, 0)),
                  pl.BlockSpec((16, D), lambda i: (0, 0)),
                  pl.BlockSpec((16, 16), lambda i: (0, 0)),
                  pl.BlockSpec((16, D), lambda i: (0, 0)),
                  pl.BlockSpec((1, 16), lambda i: (0, 0)),
                  pl.BlockSpec((1, D), lambda i: (0, 0)),
                  pl.BlockSpec((1, D), lambda i: (0, 0))],
        out_specs=pl.BlockSpec((BN, D), lambda i: (i, 0)),
        out_shape=jax.ShapeDtypeStruct((N, D), jnp.float32),
    )(dense, a0, a1, b0, b1, h,
      cm['ea'], cm['eb'], cm['ec'], cm['b6'], cm['p2'], tp['w2e'],
      cm['dmask'], tp['ln_g'], tp['ln_b'])
    return h_out


def kernel(x, edge_index, edge_weight, batch, initial_energies, params):
    p = params
    nrow = E // SD
    src2 = jnp.pad(edge_index[0].reshape(nrow, SD), ((0, NW * RW - nrow), (0, 0)))
    dst2 = jnp.pad(edge_index[1].reshape(nrow, SD), ((0, NW * RW - nrow), (0, 0)))
    ew = jnp.pad(edge_weight, ((0, EPAD - E), (0, 0)))
    emask = (jnp.arange(EPAD, dtype=jnp.int32) < E).astype(
        jnp.float32).reshape(EPAD, 1)
    z128 = jnp.zeros((NP, SW), jnp.float32)

    cm = _static_mats()
    t1 = _prep_tconv(p['t1'], p['ln1_g'], p['ln1_b'])
    t2 = _prep_tconv(p['t2'], p['ln2_g'], p['ln2_b'])
    t3 = _prep_tconv(p['t3'], p['ln3_g'], p['ln3_b'])

    h = pl.pallas_call(
        _proj_kernel,
        grid=(N // BN,),
        in_specs=[pl.BlockSpec((BN, 4), lambda i: (i, 0)),
                  pl.BlockSpec((4, D), lambda i: (0, 0)),
                  pl.BlockSpec((1, D), lambda i: (0, 0))],
        out_specs=pl.BlockSpec((BN, D), lambda i: (i, 0)),
        out_shape=jax.ShapeDtypeStruct((N, D), jnp.float32),
    )(x, p['proj_W'], p['proj_b'].reshape(1, D))

    h = _run_layer(h, src2, dst2, ew, emask, t1, cm, z128)
    h = _run_layer(h, src2, dst2, ew, emask, t2, cm, z128)
    h = _run_layer(h, src2, dst2, ew, emask, t3, cm, z128)

    sums, cnt = pl.pallas_call(
        _pool_kernel,
        grid=(N // BN,),
        in_specs=[
            pl.BlockSpec((BN, D), lambda i: (i, 0)),
            pl.BlockSpec((1, 1, BN), lambda i: (i, 0, 0)),
        ],
        out_specs=[
            pl.BlockSpec((G, D), lambda i: (0, 0)),
            pl.BlockSpec((G, 128), lambda i: (0, 0)),
        ],
        out_shape=[
            jax.ShapeDtypeStruct((G, D), jnp.float32),
            jax.ShapeDtypeStruct((G, 128), jnp.float32),
        ],
    )(h, batch.reshape(N // BN, 1, BN))

    ie = initial_energies.reshape(G, 1)
    head_args = [sums, cnt, ie,
                 p['fci_W'], p['fci_b'].reshape(1, D),
                 p['fci_g'].reshape(1, D), p['fci_bn'].reshape(1, D),
                 p['fc1_W'], p['fc1_b'].reshape(1, D),
                 p['fc1_g'].reshape(1, D), p['fc1_bn'].reshape(1, D),
                 p['fc2_W'], p['fc2_b'].reshape(1, D // 2),
                 p['fc2_g'].reshape(1, D // 2), p['fc2_bn'].reshape(1, D // 2),
                 p['fc3_W'], p['fc3_b'].reshape(1, 1)]
    out = pl.pallas_call(
        _head_kernel,
        in_specs=[_full_spec(a) for a in head_args],
        out_specs=_full_spec(jnp.zeros((G, 1))),
        out_shape=jax.ShapeDtypeStruct((G, 1), jnp.float32),
    )(*head_args)
    return out


# speedup vs baseline: 16.4373x; 1.2096x over previous
import functools

import jax
import jax.numpy as jnp
import numpy as np
from jax import lax
from jax.experimental import pallas as pl
from jax.experimental.pallas import tpu as pltpu
from jax.experimental.pallas import tpu_sc as plsc

N = 10000
E = 320000
H = 6
C = 32
D = H * C
G = 32
SCALE = 1.0 / np.sqrt(C)

NW = 32
SD = 40
CHR = 8
CH = SD * CHR
RW = 256
NCH = RW // CHR
EPW = RW * SD
EPAD = NW * EPW
NP = 10240
RT = NP // 16
GW = 256
KVW = 384
SW = 128

BN = 1000
BE = 4096


def _proj_kernel(x_ref, w_ref, b_ref, o_ref):
    o_ref[...] = jnp.dot(x_ref[...], w_ref[...],
                         preferred_element_type=jnp.float32) + b_ref[...]


def _qkvs_kernel(h_ref, wqa_ref, bqa_ref, wkv_ref, bkv_ref,
                 ws_ref, bs_ref, qa_ref, kv_ref, dn_ref):
    hb = h_ref[...]
    qa_ref[...] = jnp.dot(hb, wqa_ref[...],
                          preferred_element_type=jnp.float32) + bqa_ref[...]
    kv_ref[...] = jnp.dot(hb, wkv_ref[...],
                          preferred_element_type=jnp.float32) + bkv_ref[...]
    dn_ref[...] = jnp.dot(hb, ws_ref[...],
                          preferred_element_type=jnp.float32) + bs_ref[...]


def _alpha_kernel(qd_ref, ks_ref, w_ref, s16_ref, t16_ref, a_ref, m_ref):
    i = pl.program_id(0)
    qd = qd_ref[...]
    prod = qd * ks_ref[...]
    a1 = jnp.dot(prod, s16_ref[...], preferred_element_type=jnp.float32)
    qwe = jnp.dot(qd, t16_ref[...], preferred_element_type=jnp.float32)
    alpha = a1 + w_ref[...] * qwe
    a_ref[...] = alpha

    @pl.when(i == 0)
    def _():
        m_ref[...] = jnp.full_like(m_ref, -1e30)

    m_ref[...] = jnp.maximum(m_ref[...], jnp.max(alpha, axis=0, keepdims=True))


def _exwv_kernel(a_ref, m_ref, pmask_ref, kvs_ref, w_ref, msk_ref,
                 b6v_ref, p2_ref, ra_ref, rb_ref, rc_ref, sa_ref, sb_ref):
    ex = jnp.exp(a_ref[...] - (m_ref[...] + pmask_ref[...])) * msk_ref[...]
    ds = ex + jnp.dot(ex * w_ref[...], p2_ref[...],
                      preferred_element_type=jnp.float32)
    exb = jnp.dot(ex, b6v_ref[...], preferred_element_type=jnp.float32)
    wv = kvs_ref[...] * exb
    sa_ref[...] = jnp.dot(wv, ra_ref[...], preferred_element_type=jnp.float32)
    sb_ref[...] = (jnp.dot(wv, rb_ref[...], preferred_element_type=jnp.float32)
                   + jnp.dot(ds, rc_ref[...],
                             preferred_element_type=jnp.float32))


def _combine_kernel(dn_ref, a0_ref, a1_ref, b0_ref, b1_ref, hin_ref,
                    ea_ref, eb_ref, ec_ref, b6_ref, p2_ref, w2e_ref,
                    dmask_ref, g_ref, b_ref, o_ref):
    asum = a0_ref[...] + a1_ref[...]
    bsum = b0_ref[...] + b1_ref[...]
    num = (jnp.dot(asum, ea_ref[...], preferred_element_type=jnp.float32)
           + jnp.dot(bsum, eb_ref[...], preferred_element_type=jnp.float32))
    r = jnp.dot(bsum, ec_ref[...], preferred_element_type=jnp.float32)
    den192 = jnp.dot(r, b6_ref[...], preferred_element_type=jnp.float32)
    t1 = num / (den192 + 1e-16)
    den_shift = jnp.dot(r, p2_ref[...], preferred_element_type=jnp.float32)
    rdiv = r / (den_shift + dmask_ref[...] + 1e-16)
    t2 = jnp.dot(rdiv, w2e_ref[...], preferred_element_type=jnp.float32)
    y = dn_ref[...] + t1 + t2
    m = jnp.mean(y, axis=-1, keepdims=True)
    v = jnp.mean((y - m) ** 2, axis=-1, keepdims=True)
    yn = (y - m) / jnp.sqrt(v + 1e-5) * g_ref[...] + b_ref[...]
    o_ref[...] = hin_ref[...] + jax.nn.relu(yn)


def _pool_kernel(h_ref, b_ref, sums_ref, cnt_ref):
    i = pl.program_id(0)

    @pl.when(i == 0)
    def _():
        sums_ref[...] = jnp.zeros_like(sums_ref)
        cnt_ref[...] = jnp.zeros_like(cnt_ref)

    hb = h_ref[...]
    bb = b_ref[0]
    onehot = (bb == lax.broadcasted_iota(jnp.int32, (G, 1), 0)).astype(jnp.float32)
    sums_ref[...] += jnp.dot(onehot, hb, preferred_element_type=jnp.float32)
    cnt_ref[...] += jnp.broadcast_to(
        jnp.sum(onehot, axis=1, keepdims=True), cnt_ref.shape)


def _lnp(x, g, b):
    m = jnp.mean(x, axis=-1, keepdims=True)
    v = jnp.mean((x - m) ** 2, axis=-1, keepdims=True)
    return (x - m) / jnp.sqrt(v + 1e-5) * g + b


def _head_kernel(sums_ref, cnt_ref, ie_ref,
                 fciW_ref, fcib_ref, fcig_ref, fcibn_ref,
                 fc1W_ref, fc1b_ref, fc1g_ref, fc1bn_ref,
                 fc2W_ref, fc2b_ref, fc2g_ref, fc2bn_ref,
                 fc3W_ref, fc3b_ref, out_ref):
    cnt = cnt_ref[:, 0:1]
    gf = sums_ref[...] / jnp.maximum(cnt, 1.0)
    fi = jax.nn.relu(_lnp(jnp.dot(ie_ref[...], fciW_ref[...],
                                  preferred_element_type=jnp.float32)
                          + fcib_ref[...], fcig_ref[...], fcibn_ref[...]))
    z = jnp.concatenate([gf, fi], axis=1)
    z = jax.nn.relu(_lnp(jnp.dot(z, fc1W_ref[...],
                                 preferred_element_type=jnp.float32)
                         + fc1b_ref[...], fc1g_ref[...], fc1bn_ref[...]))
    z = jax.nn.relu(_lnp(jnp.dot(z, fc2W_ref[...],
                                 preferred_element_type=jnp.float32)
                         + fc2b_ref[...], fc2g_ref[...], fc2bn_ref[...]))
    out_ref[...] = jnp.dot(z, fc3W_ref[...],
                           preferred_element_type=jnp.float32) + fc3b_ref[...]


HC = CH // 2
HR = CHR // 2


def _gather_body(tab_hbm, idx_hbm, out_hbm, cidx, buf, gsem, wsem0, wsem1):
    width = buf.shape[1]
    c = lax.axis_index("c")
    s = lax.axis_index("s")
    w = c * 16 + s
    row0 = w * RW
    base0 = w * EPW
    wsems = (wsem0, wsem1)

    @pl.loop(0, NCH)
    def _(i):
        irow = row0 + i * CHR
        base = base0 + i * CH
        pltpu.sync_copy(idx_hbm.at[pl.ds(irow, CHR)], cidx)
        for half in range(2):
            hslice = buf.at[pl.ds(half * HC, HC)]

            @pl.when(i > 0)
            def _():
                pltpu.make_async_copy(
                    hslice, out_hbm.at[pl.ds(base + half * HC, HC)],
                    wsems[half]).wait()

            cps = [pltpu.async_copy(tab_hbm.at[cidx.at[half * HR + j]],
                                    buf.at[pl.ds((half * HR + j) * SD, SD)],
                                    gsem)
                   for j in range(HR)]
            for cp in cps:
                cp.wait()
            pltpu.async_copy(hslice, out_hbm.at[pl.ds(base + half * HC, HC)],
                             wsems[half])

    for half in range(2):
        pltpu.make_async_copy(
            buf.at[pl.ds(half * HC, HC)],
            out_hbm.at[pl.ds(base0 + (NCH - 1) * CH + half * HC, HC)],
            wsems[half]).wait()


def _scatter_body(rows_hbm, dst_hbm, zeros_hbm, out_hbm,
                  cidx, rbuf, table, asem0, asem1, lsem):
    c = lax.axis_index("c")
    s = lax.axis_index("s")
    w = c * 16 + s
    row0 = w * RW
    base0 = w * EPW
    asems = (asem0, asem1)
    pltpu.sync_copy(zeros_hbm.at[pl.ds(s * RT, RT)],
                    table.at[pl.ds(s * RT, RT)])
    plsc.subcore_barrier()

    @pl.loop(0, NCH)
    def _(i):
        irow = row0 + i * CHR
        base = base0 + i * CH
        pltpu.sync_copy(dst_hbm.at[pl.ds(irow, CHR)], cidx)
        for half in range(2):
            hslice = rbuf.at[pl.ds(half * HC, HC)]

            @pl.when(i > 0)
            def _():
                pltpu.make_async_copy(
                    rows_hbm.at[pl.ds(base + half * HC, HC)], hslice,
                    asems[half]).wait()

            pltpu.async_copy(rows_hbm.at[pl.ds(base + half * HC, HC)],
                             hslice, lsem).wait()
            for j in range(HR):
                pltpu.async_copy(rbuf.at[pl.ds((half * HR + j) * SD, SD)],
                                 table.at[cidx.at[half * HR + j]],
                                 asems[half], add=True)

    for half in range(2):
        pltpu.make_async_copy(
            rows_hbm.at[pl.ds(base0 + half * HC, HC)],
            rbuf.at[pl.ds(half * HC, HC)], asems[half]).wait()
    plsc.subcore_barrier()
    pltpu.sync_copy(table.at[pl.ds(s * RT, RT)],
                    out_hbm.at[pl.ds(c * NP + s * RT, RT)])


@functools.cache
def _sc_kernels():
    mesh = plsc.VectorSubcoreMesh(core_axis_name="c", subcore_axis_name="s")

    def gather(width):
        return pl.kernel(
            _gather_body,
            out_type=jax.ShapeDtypeStruct((EPAD, width), jnp.float32),
            mesh=mesh,
            scratch_types=[pltpu.VMEM((CHR, SD), jnp.int32),
                           pltpu.VMEM((CH, width), jnp.float32),
                           pltpu.SemaphoreType.DMA,
                           pltpu.SemaphoreType.DMA,
                           pltpu.SemaphoreType.DMA],
        )

    scatter = pl.kernel(
        _scatter_body,
        out_type=jax.ShapeDtypeStruct((2 * NP, SW), jnp.float32),
        mesh=mesh,
        scratch_types=[pltpu.VMEM((CHR, SD), jnp.int32),
                       pltpu.VMEM((CH, SW), jnp.float32),
                       pltpu.VMEM_SHARED((NP, SW), jnp.float32),
                       pltpu.SemaphoreType.DMA,
                       pltpu.SemaphoreType.DMA,
                       pltpu.SemaphoreType.DMA],
    )
    return gather(GW), gather(KVW), scatter


def _static_mats():
    s16 = np.zeros((GW, 16), np.float32)
    for j in range(D):
        s16[j, j // C] = 1.0
    t16 = np.zeros((GW, 16), np.float32)
    for h in range(H):
        t16[D + h, h] = 1.0
    b6 = np.zeros((16, D), np.float32)
    for h in range(H):
        b6[h, h * C:(h + 1) * C] = 1.0
    b6v = np.zeros((16, KVW), np.float32)
    for h in range(H):
        b6v[h, D + h * C:D + (h + 1) * C] = 1.0
    p2 = np.zeros((16, 16), np.float32)
    for h in range(H):
        p2[h, h + 6] = 1.0
    ra = np.zeros((KVW, SW), np.float32)
    ra[D:D + SW, :] = np.eye(SW, dtype=np.float32)
    rb = np.zeros((KVW, SW), np.float32)
    rb[D + SW:2 * D, :D - SW] = np.eye(D - SW, dtype=np.float32)
    rc = np.zeros((16, SW), np.float32)
    rc[:, D - SW:D - SW + 16] = np.eye(16, dtype=np.float32)
    ea = np.zeros((SW, D), np.float32)
    ea[:, :SW] = np.eye(SW, dtype=np.float32)
    eb = np.zeros((SW, D), np.float32)
    eb[:D - SW, SW:] = np.eye(D - SW, dtype=np.float32)
    ec = np.zeros((SW, 16), np.float32)
    ec[D - SW:D - SW + 16, :] = np.eye(16, dtype=np.float32)
    pmask = np.zeros((1, 16), np.float32)
    pmask[0, H:] = 1e30
    dmask = np.ones((1, 16), np.float32)
    dmask[0, H:2 * H] = 0.0
    return {k: jnp.asarray(v) for k, v in dict(
        s16=s16, t16=t16, b6=b6, b6v=b6v, p2=p2, ra=ra, rb=rb, rc=rc,
        ea=ea, eb=eb, ec=ec, pmask=pmask, dmask=dmask).items()}


_HEADMASK = (np.arange(H)[:, None] == np.arange(D)[None, :] // C
             ).astype(np.float32)


def _prep_tconv(p, ln_g, ln_b):
    we = p['We'].reshape(D)
    mask = (np.arange(D)[:, None] // C == np.arange(H)[None, :]).astype(np.float32)
    w2 = we[:, None] * mask
    padq = jnp.zeros((D, GW - D - H), jnp.float32)
    wqa = jnp.concatenate([p['Wq'] * SCALE, (p['Wq'] @ w2) * SCALE, padq], axis=1)
    bqa = jnp.concatenate([p['bq'] * SCALE, (p['bq'] @ w2) * SCALE,
                           jnp.zeros((GW - D - H,), jnp.float32)]).reshape(1, GW)
    w2e_rows = _HEADMASK * p['We'].reshape(1, D)
    w2e = jnp.concatenate([jnp.zeros((6, D), jnp.float32), w2e_rows,
                           jnp.zeros((4, D), jnp.float32)], axis=0)
    return {
        'Wqa': wqa, 'bqa': bqa,
        'Wkv': jnp.concatenate([p['Wk'], p['Wv']], axis=1),
        'bkv': jnp.concatenate([p['bk'], p['bv']]).reshape(1, KVW),
        'Ws': p['Ws'], 'bs2': p['bs'].reshape(1, D),
        'ln_g': ln_g.reshape(1, D), 'ln_b': ln_b.reshape(1, D),
        'w2e': w2e,
    }


def _full_spec(a):
    return pl.BlockSpec(a.shape, lambda *_: tuple(0 for _ in a.shape))


def _run_layer(h, src2, dst2, ew, emask, tp, cm, z128):
    qa, kv, dense = pl.pallas_call(
        _qkvs_kernel,
        grid=(N // BN,),
        in_specs=[pl.BlockSpec((BN, D), lambda i: (i, 0))] + [
            pl.BlockSpec(w.shape, lambda i: (0, 0)) for w in
            (tp['Wqa'], tp['bqa'], tp['Wkv'], tp['bkv'],
             tp['Ws'], tp['bs2'])],
        out_specs=[pl.BlockSpec((BN, GW), lambda i: (i, 0)),
                   pl.BlockSpec((BN, KVW), lambda i: (i, 0)),
                   pl.BlockSpec((BN, D), lambda i: (i, 0))],
        out_shape=[jax.ShapeDtypeStruct((N, GW), jnp.float32),
                   jax.ShapeDtypeStruct((N, KVW), jnp.float32),
                   jax.ShapeDtypeStruct((N, D), jnp.float32)],
    )(h, tp['Wqa'], tp['bqa'], tp['Wkv'], tp['bkv'], tp['Ws'], tp['bs2'])

    gather_qa, gather_kv, sc_scatter = _sc_kernels()
    qd = gather_qa(qa, dst2)
    kvs = gather_kv(kv, src2)

    alpha, m = pl.pallas_call(
        _alpha_kernel,
        grid=(EPAD // BE,),
        in_specs=[pl.BlockSpec((BE, GW), lambda i: (i, 0)),
                  pl.BlockSpec((BE, GW), lambda i: (i, 0)),
                  pl.BlockSpec((BE, 1), lambda i: (i, 0)),
                  pl.BlockSpec((GW, 16), lambda i: (0, 0)),
                  pl.BlockSpec((GW, 16), lambda i: (0, 0))],
        out_specs=[pl.BlockSpec((BE, 16), lambda i: (i, 0)),
                   pl.BlockSpec((1, 16), lambda i: (0, 0))],
        out_shape=[jax.ShapeDtypeStruct((EPAD, 16), jnp.float32),
                   jax.ShapeDtypeStruct((1, 16), jnp.float32)],
    )(qd, kvs, ew, cm['s16'], cm['t16'])

    sa, sb = pl.pallas_call(
        _exwv_kernel,
        grid=(EPAD // BE,),
        in_specs=[pl.BlockSpec((BE, 16), lambda i: (i, 0)),
                  pl.BlockSpec((1, 16), lambda i: (0, 0)),
                  pl.BlockSpec((1, 16), lambda i: (0, 0)),
                  pl.BlockSpec((BE, KVW), lambda i: (i, 0)),
                  pl.BlockSpec((BE, 1), lambda i: (i, 0)),
                  pl.BlockSpec((BE, 1), lambda i: (i, 0)),
                  pl.BlockSpec((16, KVW), lambda i: (0, 0)),
                  pl.BlockSpec((16, 16), lambda i: (0, 0)),
                  pl.BlockSpec((KVW, SW), lambda i: (0, 0)),
                  pl.BlockSpec((KVW, SW), lambda i: (0, 0)),
                  pl.BlockSpec((16, SW), lambda i: (0, 0))],
        out_specs=[pl.BlockSpec((BE, SW), lambda i: (i, 0)),
                   pl.BlockSpec((BE, SW), lambda i: (i, 0))],
        out_shape=[jax.ShapeDtypeStruct((EPAD, SW), jnp.float32),
                   jax.ShapeDtypeStruct((EPAD, SW), jnp.float32)],
    )(alpha, m, cm['pmask'], kvs, ew, emask,
      cm['b6v'], cm['p2'], cm['ra'], cm['rb'], cm['rc'])

    apart = sc_scatter(sa, dst2, z128)
    bpart = sc_scatter(sb, dst2, z128)

    a0 = lax.slice(apart, (0, 0), (N, SW))
    a1 = lax.slice(apart, (NP, 0), (NP + N, SW))
    b0 = lax.slice(bpart, (0, 0), (N, SW))
    b1 = lax.slice(bpart, (NP, 0), (NP + N, SW))

    h_out = pl.pallas_call(
        _combine_kernel,
        grid=(N // BN,),
        in_specs=[pl.BlockSpec((BN, D), lambda i: (i, 0)),
                  pl.BlockSpec((BN, SW), lambda i: (i, 0)),
                  pl.BlockSpec((BN, SW), lambda i: (i, 0)),
                  pl.BlockSpec((BN, SW), lambda i: (i, 0)),
                  pl.BlockSpec((BN, SW), lambda i: (i, 0)),
                  pl.BlockSpec((BN, D), lambda i: (i, 0)),
                  pl.BlockSpec((SW, D), lambda i: (0, 0)),
                  pl.BlockSpec((SW, D), lambda i: (0, 0)),
                  pl.BlockSpec((SW, 16), lambda i: (0, 0)),
                  pl.BlockSpec((16, D), lambda i: (0, 0)),
                  pl.BlockSpec((16, 16), lambda i: (0, 0)),
                  pl.BlockSpec((16, D), lambda i: (0, 0)),
                  pl.BlockSpec((1, 16), lambda i: (0, 0)),
                  pl.BlockSpec((1, D), lambda i: (0, 0)),
                  pl.BlockSpec((1, D), lambda i: (0, 0))],
        out_specs=pl.BlockSpec((BN, D), lambda i: (i, 0)),
        out_shape=jax.ShapeDtypeStruct((N, D), jnp.float32),
    )(dense, a0, a1, b0, b1, h,
      cm['ea'], cm['eb'], cm['ec'], cm['b6'], cm['p2'], tp['w2e'],
      cm['dmask'], tp['ln_g'], tp['ln_b'])
    return h_out


def kernel(x, edge_index, edge_weight, batch, initial_energies, params):
    p = params
    nrow = E // SD
    src2 = jnp.pad(edge_index[0].reshape(nrow, SD), ((0, NW * RW - nrow), (0, 0)))
    dst2 = jnp.pad(edge_index[1].reshape(nrow, SD), ((0, NW * RW - nrow), (0, 0)))
    ew = jnp.pad(edge_weight, ((0, EPAD - E), (0, 0)))
    emask = (jnp.arange(EPAD, dtype=jnp.int32) < E).astype(
        jnp.float32).reshape(EPAD, 1)
    z128 = jnp.zeros((NP, SW), jnp.float32)

    cm = _static_mats()
    t1 = _prep_tconv(p['t1'], p['ln1_g'], p['ln1_b'])
    t2 = _prep_tconv(p['t2'], p['ln2_g'], p['ln2_b'])
    t3 = _prep_tconv(p['t3'], p['ln3_g'], p['ln3_b'])

    h = pl.pallas_call(
        _proj_kernel,
        grid=(N // BN,),
        in_specs=[pl.BlockSpec((BN, 4), lambda i: (i, 0)),
                  pl.BlockSpec((4, D), lambda i: (0, 0)),
                  pl.BlockSpec((1, D), lambda i: (0, 0))],
        out_specs=pl.BlockSpec((BN, D), lambda i: (i, 0)),
        out_shape=jax.ShapeDtypeStruct((N, D), jnp.float32),
    )(x, p['proj_W'], p['proj_b'].reshape(1, D))

    h = _run_layer(h, src2, dst2, ew, emask, t1, cm, z128)
    h = _run_layer(h, src2, dst2, ew, emask, t2, cm, z128)
    h = _run_layer(h, src2, dst2, ew, emask, t3, cm, z128)

    sums, cnt = pl.pallas_call(
        _pool_kernel,
        grid=(N // BN,),
        in_specs=[
            pl.BlockSpec((BN, D), lambda i: (i, 0)),
            pl.BlockSpec((1, 1, BN), lambda i: (i, 0, 0)),
        ],
        out_specs=[
            pl.BlockSpec((G, D), lambda i: (0, 0)),
            pl.BlockSpec((G, 128), lambda i: (0, 0)),
        ],
        out_shape=[
            jax.ShapeDtypeStruct((G, D), jnp.float32),
            jax.ShapeDtypeStruct((G, 128), jnp.float32),
        ],
    )(h, batch.reshape(N // BN, 1, BN))

    ie = initial_energies.reshape(G, 1)
    head_args = [sums, cnt, ie,
                 p['fci_W'], p['fci_b'].reshape(1, D),
                 p['fci_g'].reshape(1, D), p['fci_bn'].reshape(1, D),
                 p['fc1_W'], p['fc1_b'].reshape(1, D),
                 p['fc1_g'].reshape(1, D), p['fc1_bn'].reshape(1, D),
                 p['fc2_W'], p['fc2_b'].reshape(1, D // 2),
                 p['fc2_g'].reshape(1, D // 2), p['fc2_bn'].reshape(1, D // 2),
                 p['fc3_W'], p['fc3_b'].reshape(1, 1)]
    out = pl.pallas_call(
        _head_kernel,
        in_specs=[_full_spec(a) for a in head_args],
        out_specs=_full_spec(jnp.zeros((G, 1))),
        out_shape=jax.ShapeDtypeStruct((G, 1), jnp.float32),
    )(*head_args)
    return out
